# Initial kernel scaffold; baseline (speedup 1.0000x reference)
import jax
import jax.numpy as jnp
from jax.experimental import pallas as pl


def kernel(x, W):
    raise NotImplementedError("write your pallas kernel here")


# speedup vs baseline: 5.1892x; 5.1892x over previous
import jax
import jax.numpy as jnp
from jax import lax
from jax.experimental import pallas as pl
from jax.experimental.pallas import tpu as pltpu
from jax.experimental.pallas import tpu_sc as plsc

D_MODEL = 128
BATCH = 4096
HIST = 200
TOTAL = BATCH * HIST

_NC, _NS = 2, 16
_NW = _NC * _NS
_PER_W = TOTAL // _NW
_CHUNK = 128
_NCHUNK = _PER_W // _CHUNK


def _emb_body(W_hbm, x_hbm, out_hbm, idx_v, rows_v, sem):
    wid = lax.axis_index("s") * _NC + lax.axis_index("c")
    base = wid * _PER_W

    def step(j, carry):
        off = base + j * _CHUNK
        pltpu.sync_copy(x_hbm.at[pl.ds(off, _CHUNK)], idx_v)
        pltpu.async_copy(W_hbm.at[idx_v], rows_v, sem).wait()
        pltpu.sync_copy(rows_v, out_hbm.at[pl.ds(off, _CHUNK)])
        return carry

    lax.fori_loop(0, _NCHUNK, step, 0)


@jax.jit
def kernel(x, W):
    xf = x.reshape(TOTAL)
    mesh = plsc.VectorSubcoreMesh(core_axis_name="c", subcore_axis_name="s")
    out = pl.kernel(
        _emb_body,
        mesh=mesh,
        out_type=jax.ShapeDtypeStruct((TOTAL, D_MODEL), jnp.float32),
        scratch_types=[
            pltpu.VMEM((_CHUNK,), jnp.int32),
            pltpu.VMEM((_CHUNK, D_MODEL), jnp.float32),
            pltpu.SemaphoreType.DMA,
        ],
    )(W, xf)
    return out.reshape(BATCH, HIST, D_MODEL)


# speedup vs baseline: 9.1580x; 1.7648x over previous
import jax
import jax.numpy as jnp
from jax import lax
from jax.experimental import pallas as pl
from jax.experimental.pallas import tpu as pltpu
from jax.experimental.pallas import tpu_sc as plsc

D_MODEL = 128
BATCH = 4096
HIST = 200
TOTAL = BATCH * HIST

_NC, _NS = 2, 16
_NW = _NC * _NS
_PER_W = TOTAL // _NW
_CHUNK = 128
_NCHUNK = _PER_W // _CHUNK
_NBUF = 5
_NSTEP = _NCHUNK // _NBUF


def _emb_body(W_hbm, x_hbm, out_hbm, idx_v, rows_v, gsems, ssems):
    wid = lax.axis_index("s") * _NC + lax.axis_index("c")
    base_chunk = wid * _NCHUNK
    base_row = wid * _PER_W

    pltpu.sync_copy(x_hbm.at[pl.ds(base_chunk, _NCHUNK)], idx_v)

    def start_gather(b, chunk):
        pltpu.async_copy(W_hbm.at[idx_v.at[chunk]], rows_v.at[b], gsems.at[b])

    def wait_gather(b):
        pltpu.make_async_copy(W_hbm.at[idx_v.at[0]], rows_v.at[b],
                              gsems.at[b]).wait()

    def start_store(b, chunk):
        pltpu.async_copy(rows_v.at[b],
                         out_hbm.at[pl.ds(base_row + chunk * _CHUNK, _CHUNK)],
                         ssems.at[b])

    def wait_store(b):
        pltpu.make_async_copy(rows_v.at[b],
                              out_hbm.at[pl.ds(base_row, _CHUNK)],
                              ssems.at[b]).wait()

    for b in range(_NBUF):
        start_gather(b, b)

    def step(i, carry):
        j = i * _NBUF
        for b in range(_NBUF):
            wait_gather(b)
            start_store(b, j + b)

        @pl.when(i < _NSTEP - 1)
        def _():
            for b in range(_NBUF):
                wait_store(b)
                start_gather(b, j + _NBUF + b)

        return carry

    lax.fori_loop(0, _NSTEP, step, 0)
    for b in range(_NBUF):
        wait_store(b)


@jax.jit
def kernel(x, W):
    xf = x.reshape(TOTAL // _CHUNK, _CHUNK)
    mesh = plsc.VectorSubcoreMesh(core_axis_name="c", subcore_axis_name="s")
    out = pl.kernel(
        _emb_body,
        mesh=mesh,
        out_type=jax.ShapeDtypeStruct((TOTAL, D_MODEL), jnp.float32),
        scratch_types=[
            pltpu.VMEM((_NCHUNK, _CHUNK), jnp.int32),
            pltpu.VMEM((_NBUF, _CHUNK, D_MODEL), jnp.float32),
            pltpu.SemaphoreType.DMA((_NBUF,)),
            pltpu.SemaphoreType.DMA((_NBUF,)),
        ],
    )(W, xf)
    return out.reshape(BATCH, HIST, D_MODEL)
